# Initial kernel scaffold; baseline (speedup 1.0000x reference)
import jax
import jax.numpy as jnp
from jax.experimental import pallas as pl


def kernel(x, edge_index, batch, W_enc, b_enc, W_layers, b_layers, gamma, beta, W_out, b_out):
    raise NotImplementedError("write your pallas kernel here")


# speedup vs baseline: 5.5430x; 5.5430x over previous
import functools

import jax
import jax.numpy as jnp
from jax import lax
from jax.experimental import pallas as pl
from jax.experimental.pallas import tpu as pltpu
from jax.experimental.pallas import tpu_sc as plsc

_N = 10000
_E = 320000
_H = 128
_G = 64
_L = 3
_K = 5
_ALPHA = 0.8

_NC = 2
_NS = 16
_NW = _NC * _NS
_EPW = _E // _NW
_CH = 80
_NCHUNK = _EPW // _CH
_RPT = 632
_RPT_LAST = _N - 15 * _RPT


def _mp_body(h_hbm, src_hbm, dst_hbm, zblk_hbm, out_hbm,
             src_v, dst_v, msg_v, zb_v, agg_sh, sem):
    c = lax.axis_index("c")
    s = lax.axis_index("s")
    wid = c * _NS + s
    pltpu.sync_copy(zblk_hbm, zb_v)
    base_r = s * _RPT
    for k in range(4):
        pltpu.sync_copy(zb_v, agg_sh.at[pl.ds(base_r + 128 * k, 128)])

    @pl.when(s < _NS - 1)
    def _():
        pltpu.sync_copy(zb_v.at[pl.ds(0, _RPT - 512)],
                        agg_sh.at[pl.ds(base_r + 512, _RPT - 512)])

    @pl.when(s == _NS - 1)
    def _():
        pltpu.sync_copy(zb_v.at[pl.ds(0, _RPT_LAST - 512)],
                        agg_sh.at[pl.ds(base_r + 512, _RPT_LAST - 512)])

    plsc.subcore_barrier()

    e0 = wid * _EPW

    def body(j, carry):
        off = e0 + j * _CH
        pltpu.sync_copy(src_hbm.at[pl.ds(off, _CH)], src_v)
        pltpu.sync_copy(dst_hbm.at[pl.ds(off, _CH)], dst_v)
        pltpu.async_copy(h_hbm.at[src_v], msg_v, sem).wait()
        pltpu.sync_copy(msg_v, agg_sh.at[dst_v], add=True)
        return carry

    lax.fori_loop(0, _NCHUNK, body, 0)
    plsc.subcore_barrier()

    @pl.when(s < _NS - 1)
    def _():
        pltpu.sync_copy(agg_sh.at[pl.ds(base_r, _RPT)],
                        out_hbm.at[c].at[pl.ds(base_r, _RPT)])

    @pl.when(s == _NS - 1)
    def _():
        pltpu.sync_copy(agg_sh.at[pl.ds(base_r, _RPT_LAST)],
                        out_hbm.at[c].at[pl.ds(base_r, _RPT_LAST)])


_mp_sc = pl.kernel(
    _mp_body,
    out_type=jax.ShapeDtypeStruct((_NC, _N, _H), jnp.float32),
    mesh=plsc.VectorSubcoreMesh(core_axis_name="c", subcore_axis_name="s"),
    scratch_types=[
        pltpu.VMEM((_CH,), jnp.int32),
        pltpu.VMEM((_CH,), jnp.int32),
        pltpu.VMEM((_CH, _H), jnp.float32),
        pltpu.VMEM((128, _H), jnp.float32),
        pltpu.VMEM_SHARED((_N, _H), jnp.float32),
        pltpu.SemaphoreType.DMA,
    ],
)


def _deg_body(dst_hbm, zblk_hbm, orow_hbm, out_hbm, dst_v, ones_v, zb_v, deg_sh):
    c = lax.axis_index("c")
    s = lax.axis_index("s")
    wid = c * _NS + s
    pltpu.sync_copy(zblk_hbm, zb_v)
    base_r = s * _RPT
    for k in range(4):
        pltpu.sync_copy(zb_v, deg_sh.at[pl.ds(base_r + 128 * k, 128)])

    @pl.when(s < _NS - 1)
    def _():
        pltpu.sync_copy(zb_v.at[pl.ds(0, _RPT - 512)],
                        deg_sh.at[pl.ds(base_r + 512, _RPT - 512)])

    @pl.when(s == _NS - 1)
    def _():
        pltpu.sync_copy(zb_v.at[pl.ds(0, _RPT_LAST - 512)],
                        deg_sh.at[pl.ds(base_r + 512, _RPT_LAST - 512)])

    pltpu.sync_copy(orow_hbm, ones_v)
    plsc.subcore_barrier()

    e0 = wid * _EPW

    def body(j, carry):
        off = e0 + j * _CH
        pltpu.sync_copy(dst_hbm.at[pl.ds(off, _CH)], dst_v)
        pltpu.sync_copy(ones_v, deg_sh.at[dst_v], add=True)
        return carry

    lax.fori_loop(0, _NCHUNK, body, 0)
    plsc.subcore_barrier()

    @pl.when(s < _NS - 1)
    def _():
        pltpu.sync_copy(deg_sh.at[pl.ds(base_r, _RPT)],
                        out_hbm.at[c].at[pl.ds(base_r, _RPT)])

    @pl.when(s == _NS - 1)
    def _():
        pltpu.sync_copy(deg_sh.at[pl.ds(base_r, _RPT_LAST)],
                        out_hbm.at[c].at[pl.ds(base_r, _RPT_LAST)])


_deg_sc = pl.kernel(
    _deg_body,
    out_type=jax.ShapeDtypeStruct((_NC, _N, 16), jnp.float32),
    mesh=plsc.VectorSubcoreMesh(core_axis_name="c", subcore_axis_name="s"),
    scratch_types=[
        pltpu.VMEM((_CH,), jnp.int32),
        pltpu.VMEM((_CH, 16), jnp.float32),
        pltpu.VMEM((128, 16), jnp.float32),
        pltpu.VMEM_SHARED((_N, 16), jnp.float32),
    ],
)


def _enc_body(x_ref, w_ref, b_ref, o_ref):
    o_ref[...] = jnp.dot(x_ref[...], w_ref[...],
                         preferred_element_type=jnp.float32) + b_ref[...]


def _enc(x, w, b):
    return pl.pallas_call(
        _enc_body,
        out_shape=jax.ShapeDtypeStruct((_N, _H), jnp.float32),
    )(x, w, b.reshape(1, _H))


def _gin_body(h_ref, p_ref, w_ref, b_ref, g_ref, be_ref, o_ref):
    z = h_ref[...] + p_ref[0] + p_ref[1]
    r = jax.nn.relu(jnp.dot(z, w_ref[...],
                            preferred_element_type=jnp.float32) + b_ref[...])
    mu = jnp.mean(r, axis=0, keepdims=True)
    var = jnp.mean((r - mu) ** 2, axis=0, keepdims=True)
    o_ref[...] = (r - mu) / jnp.sqrt(var + 1e-5) * g_ref[...] + be_ref[...]


def _gin(h, parts, w, b, g, be):
    return pl.pallas_call(
        _gin_body,
        out_shape=jax.ShapeDtypeStruct((_N, _H), jnp.float32),
    )(h, parts, w, b.reshape(1, _H), g.reshape(1, _H), be.reshape(1, _H))


def _appnp_setup_body(dp_ref, h_ref, isd_ref, sn_ref, hs_ref):
    deg = dp_ref[0][:, :1] + dp_ref[1][:, :1] + 1.0
    isd = 1.0 / jnp.sqrt(deg)
    isd_ref[...] = isd
    sn_ref[...] = 1.0 / deg
    hs_ref[...] = h_ref[...] * isd


def _appnp_setup(deg_parts, h):
    return pl.pallas_call(
        _appnp_setup_body,
        out_shape=(
            jax.ShapeDtypeStruct((_N, 1), jnp.float32),
            jax.ShapeDtypeStruct((_N, 1), jnp.float32),
            jax.ShapeDtypeStruct((_N, _H), jnp.float32),
        ),
    )(deg_parts, h)


def _appnp_update_body(p_ref, h_ref, h0_ref, isd_ref, sn_ref, hn_ref, hsn_ref):
    agg = p_ref[0] + p_ref[1]
    prop = agg * isd_ref[...] + h_ref[...] * sn_ref[...]
    hn = (1.0 - _ALPHA) * prop + _ALPHA * h0_ref[...]
    hn_ref[...] = hn
    hsn_ref[...] = hn * isd_ref[...]


def _appnp_update(parts, h, h0, isd, sn):
    return pl.pallas_call(
        _appnp_update_body,
        out_shape=(
            jax.ShapeDtypeStruct((_N, _H), jnp.float32),
            jax.ShapeDtypeStruct((_N, _H), jnp.float32),
        ),
    )(parts, h, h0, isd, sn)


def _pool_body(h_ref, b_ref, w_ref, bo_ref, o_ref):
    gids = lax.broadcasted_iota(jnp.int32, (1, _G), 1)
    onehot = (b_ref[...] == gids).astype(jnp.float32)
    sums = lax.dot_general(onehot, h_ref[...], (((0,), (0,)), ((), ())),
                           preferred_element_type=jnp.float32)
    counts = jnp.sum(onehot, axis=0, keepdims=True)
    pooled = sums / jnp.maximum(counts, 1.0).T
    o_ref[...] = jnp.dot(pooled, w_ref[...],
                         preferred_element_type=jnp.float32) + bo_ref[...]


def _pool_out(h, batch, w, b):
    return pl.pallas_call(
        _pool_body,
        out_shape=jax.ShapeDtypeStruct((_G, _H), jnp.float32),
    )(h, batch.reshape(_N, 1), w, b.reshape(1, _H))


def _mp_dbg(h, src, dst):
    agg = jax.ops.segment_sum(jnp.take(h, src, axis=0), dst, num_segments=_N)
    return jnp.stack([agg, jnp.zeros_like(agg)])


def _deg_dbg(dst):
    d = jnp.zeros((_N,), jnp.float32).at[dst].add(1.0)
    d = jnp.broadcast_to(d[:, None], (_N, 16))
    return jnp.stack([d, jnp.zeros_like(d)])


def kernel(x, edge_index, batch, W_enc, b_enc, W_layers, b_layers,
           gamma, beta, W_out, b_out):
    src = edge_index[0]
    dst = edge_index[1]
    zblk = jnp.zeros((128, _H), jnp.float32)
    zrow = jnp.zeros((128, 16), jnp.float32)
    orow = jnp.ones((_CH, 16), jnp.float32)

    h = _enc(x, W_enc, b_enc)
    for i in range(_L):
        parts = _mp_sc(h, src, dst, zblk)
        h = _gin(h, parts, W_layers[i], b_layers[i], gamma[i], beta[i])

    ones_nh = jnp.ones((_N, _H), jnp.float32)
    deg_parts = _mp_sc(ones_nh, src, dst, zblk)[:, :, :16]
    isd, sn, hs = _appnp_setup(deg_parts, h)
    h0 = h
    for _ in range(_K):
        parts = _mp_sc(hs, src, dst, zblk)
        h, hs = _appnp_update(parts, h, h0, isd, sn)

    return _pool_out(h, batch, W_out, b_out)


# speedup vs baseline: 13.1571x; 2.3736x over previous
import jax
import jax.numpy as jnp
from jax import lax
from jax.experimental import pallas as pl
from jax.experimental.pallas import tpu as pltpu
from jax.experimental.pallas import tpu_sc as plsc

_N = 10000
_E = 320000
_H = 128
_G = 64
_L = 3
_K = 5
_ALPHA = 0.8

_NC = 2
_NS = 16
_NW = _NC * _NS
_EPW = _E // _NW
_CH = 80
_NCHUNK = _EPW // _CH
_NPAIR = (_NCHUNK - 1) // 2
_RPT = 632
_RPT_LAST = _N - 15 * _RPT


def _zero_my_slice(zb_v, agg_sh, s):
    base_r = s * _RPT
    for k in range(6):
        pltpu.sync_copy(zb_v, agg_sh.at[pl.ds(base_r + _CH * k, _CH)])

    @pl.when(s < _NS - 1)
    def _():
        pltpu.sync_copy(zb_v, agg_sh.at[pl.ds(base_r + 480, 80)])
        pltpu.sync_copy(zb_v.at[pl.ds(0, 72)],
                        agg_sh.at[pl.ds(base_r + 560, 72)])

    @pl.when(s == _NS - 1)
    def _():
        pltpu.sync_copy(zb_v.at[pl.ds(0, 40)],
                        agg_sh.at[pl.ds(base_r + 480, 40)])


def _flush_my_slice(agg_sh, out_hbm, c, s):
    base_r = s * _RPT

    @pl.when(s < _NS - 1)
    def _():
        pltpu.sync_copy(agg_sh.at[pl.ds(base_r, _RPT)],
                        out_hbm.at[c].at[pl.ds(base_r, _RPT)])

    @pl.when(s == _NS - 1)
    def _():
        pltpu.sync_copy(agg_sh.at[pl.ds(base_r, _RPT_LAST)],
                        out_hbm.at[c].at[pl.ds(base_r, _RPT_LAST)])


def _mp_body(h_hbm, src_hbm, dst_hbm, zblk_hbm, out_hbm,
             src0_v, src1_v, dst0_v, dst1_v, msg0_v, msg1_v, agg_sh,
             issem0, issem1, idsem0, idsem1, gsem0, gsem1, ssem0, ssem1):
    c = lax.axis_index("c")
    s = lax.axis_index("s")
    wid = c * _NS + s
    e0 = wid * _EPW

    def load_src(j, buf, sem):
        pltpu.async_copy(src_hbm.at[pl.ds(e0 + j * _CH, _CH)], buf, sem)

    def wait_src(j, buf, sem):
        pltpu.make_async_copy(src_hbm.at[pl.ds(e0 + j * _CH, _CH)], buf,
                              sem).wait()

    def load_dst(j, buf, sem):
        pltpu.async_copy(dst_hbm.at[pl.ds(e0 + j * _CH, _CH)], buf, sem)

    def wait_dst(j, buf, sem):
        pltpu.make_async_copy(dst_hbm.at[pl.ds(e0 + j * _CH, _CH)], buf,
                              sem).wait()

    def gather(buf_idx, msg, sem):
        pltpu.async_copy(h_hbm.at[buf_idx], msg, sem)

    def wait_gather(buf_idx, msg, sem):
        pltpu.make_async_copy(h_hbm.at[buf_idx], msg, sem).wait()

    def scatter(msg, buf_idx, sem):
        pltpu.async_copy(msg, agg_sh.at[buf_idx], sem, add=True)

    def wait_scatter(msg, buf_idx, sem):
        pltpu.make_async_copy(msg, agg_sh.at[buf_idx], sem).wait()

    load_src(0, src0_v, issem0)
    load_dst(0, dst0_v, idsem0)
    load_src(1, src1_v, issem1)
    load_dst(1, dst1_v, idsem1)
    pltpu.sync_copy(zblk_hbm, msg0_v)
    _zero_my_slice(msg0_v, agg_sh, s)
    plsc.subcore_barrier()

    wait_src(0, src0_v, issem0)
    gather(src0_v, msg0_v, gsem0)

    def pair(i, carry):
        j0 = 2 * i
        j1 = j0 + 1

        @pl.when(i > 0)
        def _():
            wait_scatter(msg1_v, dst1_v, ssem1)
            load_dst(j1, dst1_v, idsem1)

        wait_src(j1, src1_v, issem1)
        gather(src1_v, msg1_v, gsem1)
        wait_gather(src0_v, msg0_v, gsem0)
        load_src(j0 + 2, src0_v, issem0)
        wait_dst(j0, dst0_v, idsem0)
        scatter(msg0_v, dst0_v, ssem0)
        wait_scatter(msg0_v, dst0_v, ssem0)
        load_dst(j0 + 2, dst0_v, idsem0)
        wait_src(j0 + 2, src0_v, issem0)
        gather(src0_v, msg0_v, gsem0)

        @pl.when(i < _NPAIR - 1)
        def _():
            load_src(j1 + 2, src1_v, issem1)

        wait_gather(src1_v, msg1_v, gsem1)
        wait_dst(j1, dst1_v, idsem1)
        scatter(msg1_v, dst1_v, ssem1)
        return carry

    lax.fori_loop(0, _NPAIR, pair, 0)

    last = _NCHUNK - 1
    wait_gather(src0_v, msg0_v, gsem0)
    wait_dst(last, dst0_v, idsem0)
    scatter(msg0_v, dst0_v, ssem0)
    wait_scatter(msg0_v, dst0_v, ssem0)
    wait_scatter(msg1_v, dst1_v, ssem1)
    plsc.subcore_barrier()
    _flush_my_slice(agg_sh, out_hbm, c, s)


_mp_sc = pl.kernel(
    _mp_body,
    out_type=jax.ShapeDtypeStruct((_NC, _N, _H), jnp.float32),
    mesh=plsc.VectorSubcoreMesh(core_axis_name="c", subcore_axis_name="s"),
    scratch_types=[
        pltpu.VMEM((_CH,), jnp.int32),
        pltpu.VMEM((_CH,), jnp.int32),
        pltpu.VMEM((_CH,), jnp.int32),
        pltpu.VMEM((_CH,), jnp.int32),
        pltpu.VMEM((_CH, _H), jnp.float32),
        pltpu.VMEM((_CH, _H), jnp.float32),
        pltpu.VMEM_SHARED((_N, _H), jnp.float32),
        pltpu.SemaphoreType.DMA,
        pltpu.SemaphoreType.DMA,
        pltpu.SemaphoreType.DMA,
        pltpu.SemaphoreType.DMA,
        pltpu.SemaphoreType.DMA,
        pltpu.SemaphoreType.DMA,
        pltpu.SemaphoreType.DMA,
        pltpu.SemaphoreType.DMA,
    ],
)


def _enc_body(x_ref, w_ref, b_ref, o_ref):
    o_ref[...] = jnp.dot(x_ref[...], w_ref[...],
                         preferred_element_type=jnp.float32) + b_ref[...]


def _enc(x, w, b):
    return pl.pallas_call(
        _enc_body,
        out_shape=jax.ShapeDtypeStruct((_N, _H), jnp.float32),
    )(x, w, b.reshape(1, _H))


def _gin_body(h_ref, p_ref, w_ref, b_ref, g_ref, be_ref, o_ref):
    z = h_ref[...] + p_ref[0] + p_ref[1]
    r = jax.nn.relu(jnp.dot(z, w_ref[...],
                            preferred_element_type=jnp.float32) + b_ref[...])
    mu = jnp.mean(r, axis=0, keepdims=True)
    var = jnp.mean((r - mu) ** 2, axis=0, keepdims=True)
    o_ref[...] = (r - mu) / jnp.sqrt(var + 1e-5) * g_ref[...] + be_ref[...]


def _gin(h, parts, w, b, g, be):
    return pl.pallas_call(
        _gin_body,
        out_shape=jax.ShapeDtypeStruct((_N, _H), jnp.float32),
    )(h, parts, w, b.reshape(1, _H), g.reshape(1, _H), be.reshape(1, _H))


def _appnp_setup_body(dp_ref, h_ref, isd_ref, sn_ref, hs_ref):
    deg = dp_ref[0][:, :1] + dp_ref[1][:, :1] + 1.0
    isd = 1.0 / jnp.sqrt(deg)
    isd_ref[...] = isd
    sn_ref[...] = 1.0 / deg
    hs_ref[...] = h_ref[...] * isd


def _appnp_setup(deg_parts, h):
    return pl.pallas_call(
        _appnp_setup_body,
        out_shape=(
            jax.ShapeDtypeStruct((_N, 1), jnp.float32),
            jax.ShapeDtypeStruct((_N, 1), jnp.float32),
            jax.ShapeDtypeStruct((_N, _H), jnp.float32),
        ),
    )(deg_parts, h)


def _appnp_update_body(p_ref, h_ref, h0_ref, isd_ref, sn_ref, hn_ref, hsn_ref):
    agg = p_ref[0] + p_ref[1]
    prop = agg * isd_ref[...] + h_ref[...] * sn_ref[...]
    hn = (1.0 - _ALPHA) * prop + _ALPHA * h0_ref[...]
    hn_ref[...] = hn
    hsn_ref[...] = hn * isd_ref[...]


def _appnp_update(parts, h, h0, isd, sn):
    return pl.pallas_call(
        _appnp_update_body,
        out_shape=(
            jax.ShapeDtypeStruct((_N, _H), jnp.float32),
            jax.ShapeDtypeStruct((_N, _H), jnp.float32),
        ),
    )(parts, h, h0, isd, sn)


def _pool_body(h_ref, b_ref, w_ref, bo_ref, o_ref):
    gids = lax.broadcasted_iota(jnp.int32, (1, _G), 1)
    onehot = (b_ref[...] == gids).astype(jnp.float32)
    sums = lax.dot_general(onehot, h_ref[...], (((0,), (0,)), ((), ())),
                           preferred_element_type=jnp.float32)
    counts = jnp.sum(onehot, axis=0, keepdims=True)
    pooled = sums / jnp.maximum(counts, 1.0).T
    o_ref[...] = jnp.dot(pooled, w_ref[...],
                         preferred_element_type=jnp.float32) + bo_ref[...]


def _pool_out(h, batch, w, b):
    return pl.pallas_call(
        _pool_body,
        out_shape=jax.ShapeDtypeStruct((_G, _H), jnp.float32),
    )(h, batch.reshape(_N, 1), w, b.reshape(1, _H))


def kernel(x, edge_index, batch, W_enc, b_enc, W_layers, b_layers,
           gamma, beta, W_out, b_out):
    src = edge_index[0]
    dst = edge_index[1]
    zblk = jnp.zeros((_CH, _H), jnp.float32)

    h = _enc(x, W_enc, b_enc)
    for i in range(_L):
        parts = _mp_sc(h, src, dst, zblk)
        h = _gin(h, parts, W_layers[i], b_layers[i], gamma[i], beta[i])

    ones_nh = jnp.ones((_N, _H), jnp.float32)
    deg_parts = _mp_sc(ones_nh, src, dst, zblk)
    isd, sn, hs = _appnp_setup(deg_parts, h)
    h0 = h
    for _ in range(_K):
        parts = _mp_sc(hs, src, dst, zblk)
        h, hs = _appnp_update(parts, h, h0, isd, sn)

    return _pool_out(h, batch, W_out, b_out)


# speedup vs baseline: 13.6366x; 1.0364x over previous
import jax
import jax.numpy as jnp
from jax import lax
from jax.experimental import pallas as pl
from jax.experimental.pallas import tpu as pltpu
from jax.experimental.pallas import tpu_sc as plsc

_N = 10000
_E = 320000
_H = 128
_G = 64
_L = 3
_K = 5
_ALPHA = 0.8

_NC = 2
_NS = 16
_NW = _NC * _NS
_EPW = _E // _NW
_CH = 80
_NCHUNK = _EPW // _CH
_NPAIR = (_NCHUNK - 1) // 2
_RPT = 632
_RPT_LAST = _N - 15 * _RPT


def _zero_my_slice(zb_v, agg_sh, s):
    base_r = s * _RPT
    for k in range(6):
        pltpu.sync_copy(zb_v, agg_sh.at[pl.ds(base_r + _CH * k, _CH)])

    @pl.when(s < _NS - 1)
    def _():
        pltpu.sync_copy(zb_v, agg_sh.at[pl.ds(base_r + 480, 80)])
        pltpu.sync_copy(zb_v.at[pl.ds(0, 72)],
                        agg_sh.at[pl.ds(base_r + 560, 72)])

    @pl.when(s == _NS - 1)
    def _():
        pltpu.sync_copy(zb_v.at[pl.ds(0, 40)],
                        agg_sh.at[pl.ds(base_r + 480, 40)])


def _flush_my_slice(agg_sh, out_hbm, c, s):
    base_r = s * _RPT

    @pl.when(s < _NS - 1)
    def _():
        pltpu.sync_copy(agg_sh.at[pl.ds(base_r, _RPT)],
                        out_hbm.at[c].at[pl.ds(base_r, _RPT)])

    @pl.when(s == _NS - 1)
    def _():
        pltpu.sync_copy(agg_sh.at[pl.ds(base_r, _RPT_LAST)],
                        out_hbm.at[c].at[pl.ds(base_r, _RPT_LAST)])


def _mp_body(h_hbm, src_hbm, dst_hbm, zblk_hbm, out_hbm,
             src0_v, src1_v, dst0_v, dst1_v, msg0_v, msg1_v, agg_sh,
             issem0, issem1, idsem0, idsem1, gsem0, gsem1, ssem0, ssem1):
    c = lax.axis_index("c")
    s = lax.axis_index("s")
    wid = c * _NS + s
    e0 = wid * _EPW

    def load_src(j, buf, sem):
        pltpu.async_copy(src_hbm.at[pl.ds(e0 + j * _CH, _CH)], buf, sem)

    def wait_src(j, buf, sem):
        pltpu.make_async_copy(src_hbm.at[pl.ds(e0 + j * _CH, _CH)], buf,
                              sem).wait()

    def load_dst(j, buf, sem):
        pltpu.async_copy(dst_hbm.at[pl.ds(e0 + j * _CH, _CH)], buf, sem)

    def wait_dst(j, buf, sem):
        pltpu.make_async_copy(dst_hbm.at[pl.ds(e0 + j * _CH, _CH)], buf,
                              sem).wait()

    def gather(buf_idx, msg, sem):
        pltpu.async_copy(h_hbm.at[buf_idx], msg, sem)

    def wait_gather(buf_idx, msg, sem):
        pltpu.make_async_copy(h_hbm.at[buf_idx], msg, sem).wait()

    def scatter(msg, buf_idx, sem):
        pltpu.async_copy(msg, agg_sh.at[buf_idx], sem, add=True)

    def wait_scatter(msg, buf_idx, sem):
        pltpu.make_async_copy(msg, agg_sh.at[buf_idx], sem).wait()

    load_src(0, src0_v, issem0)
    load_dst(0, dst0_v, idsem0)
    load_src(1, src1_v, issem1)
    load_dst(1, dst1_v, idsem1)
    pltpu.sync_copy(zblk_hbm, msg0_v)
    _zero_my_slice(msg0_v, agg_sh, s)
    plsc.subcore_barrier()

    wait_src(0, src0_v, issem0)
    gather(src0_v, msg0_v, gsem0)

    def pair(i, carry):
        j0 = 2 * i
        j1 = j0 + 1

        @pl.when(i > 0)
        def _():
            wait_scatter(msg1_v, dst1_v, ssem1)
            load_dst(j1, dst1_v, idsem1)

        wait_src(j1, src1_v, issem1)
        gather(src1_v, msg1_v, gsem1)
        wait_gather(src0_v, msg0_v, gsem0)
        load_src(j0 + 2, src0_v, issem0)
        wait_dst(j0, dst0_v, idsem0)
        scatter(msg0_v, dst0_v, ssem0)
        wait_scatter(msg0_v, dst0_v, ssem0)
        load_dst(j0 + 2, dst0_v, idsem0)
        wait_src(j0 + 2, src0_v, issem0)
        gather(src0_v, msg0_v, gsem0)

        @pl.when(i < _NPAIR - 1)
        def _():
            load_src(j1 + 2, src1_v, issem1)

        wait_gather(src1_v, msg1_v, gsem1)
        wait_dst(j1, dst1_v, idsem1)
        scatter(msg1_v, dst1_v, ssem1)
        return carry

    lax.fori_loop(0, _NPAIR, pair, 0)

    last = _NCHUNK - 1
    wait_gather(src0_v, msg0_v, gsem0)
    wait_dst(last, dst0_v, idsem0)
    scatter(msg0_v, dst0_v, ssem0)
    wait_scatter(msg0_v, dst0_v, ssem0)
    wait_scatter(msg1_v, dst1_v, ssem1)
    plsc.subcore_barrier()
    _flush_my_slice(agg_sh, out_hbm, c, s)


_mp_sc = pl.kernel(
    _mp_body,
    out_type=jax.ShapeDtypeStruct((_NC, _N, _H), jnp.float32),
    mesh=plsc.VectorSubcoreMesh(core_axis_name="c", subcore_axis_name="s"),
    scratch_types=[
        pltpu.VMEM((_CH,), jnp.int32),
        pltpu.VMEM((_CH,), jnp.int32),
        pltpu.VMEM((_CH,), jnp.int32),
        pltpu.VMEM((_CH,), jnp.int32),
        pltpu.VMEM((_CH, _H), jnp.float32),
        pltpu.VMEM((_CH, _H), jnp.float32),
        pltpu.VMEM_SHARED((_N, _H), jnp.float32),
        pltpu.SemaphoreType.DMA,
        pltpu.SemaphoreType.DMA,
        pltpu.SemaphoreType.DMA,
        pltpu.SemaphoreType.DMA,
        pltpu.SemaphoreType.DMA,
        pltpu.SemaphoreType.DMA,
        pltpu.SemaphoreType.DMA,
        pltpu.SemaphoreType.DMA,
    ],
)


def _deg_body(dst_hbm, zblk_hbm, oblk_hbm, out_hbm,
              dst0_v, dst1_v, buf_v, agg_sh,
              idsem0, idsem1, ssem0, ssem1):
    c = lax.axis_index("c")
    s = lax.axis_index("s")
    wid = c * _NS + s
    e0 = wid * _EPW

    def load_dst(j, buf, sem):
        pltpu.async_copy(dst_hbm.at[pl.ds(e0 + j * _CH, _CH)], buf, sem)

    def wait_dst(j, buf, sem):
        pltpu.make_async_copy(dst_hbm.at[pl.ds(e0 + j * _CH, _CH)], buf,
                              sem).wait()

    def scatter(buf_idx, sem):
        pltpu.async_copy(buf_v, agg_sh.at[buf_idx], sem, add=True)

    def wait_scatter(buf_idx, sem):
        pltpu.make_async_copy(buf_v, agg_sh.at[buf_idx], sem).wait()

    load_dst(0, dst0_v, idsem0)
    load_dst(1, dst1_v, idsem1)
    pltpu.sync_copy(zblk_hbm, buf_v)
    _zero_my_slice(buf_v, agg_sh, s)
    plsc.subcore_barrier()
    pltpu.sync_copy(oblk_hbm, buf_v)

    def pair(i, carry):
        j0 = 2 * i
        j1 = j0 + 1
        wait_dst(j0, dst0_v, idsem0)
        scatter(dst0_v, ssem0)
        wait_scatter(dst0_v, ssem0)
        load_dst(j0 + 2, dst0_v, idsem0)
        wait_dst(j1, dst1_v, idsem1)
        scatter(dst1_v, ssem1)
        wait_scatter(dst1_v, ssem1)

        @pl.when(i < _NPAIR - 1)
        def _():
            load_dst(j1 + 2, dst1_v, idsem1)

        return carry

    lax.fori_loop(0, _NPAIR, pair, 0)

    last = _NCHUNK - 1
    wait_dst(last, dst0_v, idsem0)
    scatter(dst0_v, ssem0)
    wait_scatter(dst0_v, ssem0)
    plsc.subcore_barrier()
    _flush_my_slice(agg_sh, out_hbm, c, s)


_deg_sc = pl.kernel(
    _deg_body,
    out_type=jax.ShapeDtypeStruct((_NC, _N, _H), jnp.float32),
    mesh=plsc.VectorSubcoreMesh(core_axis_name="c", subcore_axis_name="s"),
    scratch_types=[
        pltpu.VMEM((_CH,), jnp.int32),
        pltpu.VMEM((_CH,), jnp.int32),
        pltpu.VMEM((_CH, _H), jnp.float32),
        pltpu.VMEM_SHARED((_N, _H), jnp.float32),
        pltpu.SemaphoreType.DMA,
        pltpu.SemaphoreType.DMA,
        pltpu.SemaphoreType.DMA,
        pltpu.SemaphoreType.DMA,
    ],
)


def _enc_body(x_ref, w_ref, b_ref, o_ref):
    o_ref[...] = jnp.dot(x_ref[...], w_ref[...],
                         preferred_element_type=jnp.float32) + b_ref[...]


def _enc(x, w, b):
    return pl.pallas_call(
        _enc_body,
        out_shape=jax.ShapeDtypeStruct((_N, _H), jnp.float32),
    )(x, w, b.reshape(1, _H))


def _gin_body(h_ref, p_ref, w_ref, b_ref, g_ref, be_ref, o_ref):
    z = h_ref[...] + p_ref[0] + p_ref[1]
    r = jax.nn.relu(jnp.dot(z, w_ref[...],
                            preferred_element_type=jnp.float32) + b_ref[...])
    mu = jnp.mean(r, axis=0, keepdims=True)
    var = jnp.mean((r - mu) ** 2, axis=0, keepdims=True)
    o_ref[...] = (r - mu) / jnp.sqrt(var + 1e-5) * g_ref[...] + be_ref[...]


def _gin(h, parts, w, b, g, be):
    return pl.pallas_call(
        _gin_body,
        out_shape=jax.ShapeDtypeStruct((_N, _H), jnp.float32),
    )(h, parts, w, b.reshape(1, _H), g.reshape(1, _H), be.reshape(1, _H))


def _appnp_setup_body(dp_ref, h_ref, isd_ref, sn_ref, hs_ref):
    deg = dp_ref[0][:, :1] + dp_ref[1][:, :1] + 1.0
    isd = 1.0 / jnp.sqrt(deg)
    isd_ref[...] = isd
    sn_ref[...] = 1.0 / deg
    hs_ref[...] = h_ref[...] * isd


def _appnp_setup(deg_parts, h):
    return pl.pallas_call(
        _appnp_setup_body,
        out_shape=(
            jax.ShapeDtypeStruct((_N, 1), jnp.float32),
            jax.ShapeDtypeStruct((_N, 1), jnp.float32),
            jax.ShapeDtypeStruct((_N, _H), jnp.float32),
        ),
    )(deg_parts, h)


def _appnp_update_body(p_ref, h_ref, h0_ref, isd_ref, sn_ref, hn_ref, hsn_ref):
    agg = p_ref[0] + p_ref[1]
    prop = agg * isd_ref[...] + h_ref[...] * sn_ref[...]
    hn = (1.0 - _ALPHA) * prop + _ALPHA * h0_ref[...]
    hn_ref[...] = hn
    hsn_ref[...] = hn * isd_ref[...]


def _appnp_update(parts, h, h0, isd, sn):
    return pl.pallas_call(
        _appnp_update_body,
        out_shape=(
            jax.ShapeDtypeStruct((_N, _H), jnp.float32),
            jax.ShapeDtypeStruct((_N, _H), jnp.float32),
        ),
    )(parts, h, h0, isd, sn)


def _pool_body(h_ref, b_ref, w_ref, bo_ref, o_ref):
    gids = lax.broadcasted_iota(jnp.int32, (1, _G), 1)
    onehot = (b_ref[...] == gids).astype(jnp.float32)
    sums = lax.dot_general(onehot, h_ref[...], (((0,), (0,)), ((), ())),
                           preferred_element_type=jnp.float32)
    counts = jnp.sum(onehot, axis=0, keepdims=True)
    pooled = sums / jnp.maximum(counts, 1.0).T
    o_ref[...] = jnp.dot(pooled, w_ref[...],
                         preferred_element_type=jnp.float32) + bo_ref[...]


def _pool_out(h, batch, w, b):
    return pl.pallas_call(
        _pool_body,
        out_shape=jax.ShapeDtypeStruct((_G, _H), jnp.float32),
    )(h, batch.reshape(_N, 1), w, b.reshape(1, _H))


def kernel(x, edge_index, batch, W_enc, b_enc, W_layers, b_layers,
           gamma, beta, W_out, b_out):
    src = edge_index[0]
    dst = edge_index[1]
    zblk = jnp.zeros((_CH, _H), jnp.float32)
    oblk = jnp.ones((_CH, _H), jnp.float32)

    deg_parts = _deg_sc(dst, zblk, oblk)
    h = _enc(x, W_enc, b_enc)
    for i in range(_L):
        parts = _mp_sc(h, src, dst, zblk)
        h = _gin(h, parts, W_layers[i], b_layers[i], gamma[i], beta[i])

    isd, sn, hs = _appnp_setup(deg_parts, h)
    h0 = h
    for _ in range(_K):
        parts = _mp_sc(hs, src, dst, zblk)
        h, hs = _appnp_update(parts, h, h0, isd, sn)

    return _pool_out(h, batch, W_out, b_out)


# speedup vs baseline: 14.2257x; 1.0432x over previous
import jax
import jax.numpy as jnp
from jax import lax
from jax.experimental import pallas as pl
from jax.experimental.pallas import tpu as pltpu
from jax.experimental.pallas import tpu_sc as plsc

_N = 10000
_E = 320000
_H = 128
_G = 64
_L = 3
_K = 5
_ALPHA = 0.8

_NC = 2
_NS = 16
_NW = _NC * _NS
_EPW = _E // _NW
_CH = 80
_NCHUNK = _EPW // _CH
_NPAIR = (_NCHUNK - 1) // 2
_RPT = 632
_RPT_LAST = _N - 15 * _RPT


def _zero_my_slice(zb_v, agg_sh, s):
    base_r = s * _RPT
    for k in range(6):
        pltpu.sync_copy(zb_v, agg_sh.at[pl.ds(base_r + _CH * k, _CH)])

    @pl.when(s < _NS - 1)
    def _():
        pltpu.sync_copy(zb_v, agg_sh.at[pl.ds(base_r + 480, 80)])
        pltpu.sync_copy(zb_v.at[pl.ds(0, 72)],
                        agg_sh.at[pl.ds(base_r + 560, 72)])

    @pl.when(s == _NS - 1)
    def _():
        pltpu.sync_copy(zb_v.at[pl.ds(0, 40)],
                        agg_sh.at[pl.ds(base_r + 480, 40)])


def _flush_my_slice(agg_sh, out_hbm, c, s):
    base_r = s * _RPT

    @pl.when(s < _NS - 1)
    def _():
        pltpu.sync_copy(agg_sh.at[pl.ds(base_r, _RPT)],
                        out_hbm.at[c].at[pl.ds(base_r, _RPT)])

    @pl.when(s == _NS - 1)
    def _():
        pltpu.sync_copy(agg_sh.at[pl.ds(base_r, _RPT_LAST)],
                        out_hbm.at[c].at[pl.ds(base_r, _RPT_LAST)])


_B = 4
_NGRP = (_NCHUNK - 1) // _B


def _mp_body(h_hbm, src_hbm, dst_hbm, zblk_hbm, out_hbm,
             s0_v, s1_v, s2_v, s3_v, d0_v, d1_v, d2_v, d3_v,
             m0_v, m1_v, m2_v, m3_v, agg_sh,
             is0, is1, is2, is3, id0, id1, id2, id3,
             gs0, gs1, gs2, gs3, ss0, ss1, ss2, ss3):
    src_v = [s0_v, s1_v, s2_v, s3_v]
    dst_v = [d0_v, d1_v, d2_v, d3_v]
    msg_v = [m0_v, m1_v, m2_v, m3_v]
    issem = [is0, is1, is2, is3]
    idsem = [id0, id1, id2, id3]
    gsem = [gs0, gs1, gs2, gs3]
    ssem = [ss0, ss1, ss2, ss3]

    c = lax.axis_index("c")
    s = lax.axis_index("s")
    wid = c * _NS + s
    e0 = wid * _EPW

    def load_src(j, b):
        pltpu.async_copy(src_hbm.at[pl.ds(e0 + j * _CH, _CH)], src_v[b],
                         issem[b])

    def wait_src(j, b):
        pltpu.make_async_copy(src_hbm.at[pl.ds(e0 + j * _CH, _CH)], src_v[b],
                              issem[b]).wait()

    def load_dst(j, b):
        pltpu.async_copy(dst_hbm.at[pl.ds(e0 + j * _CH, _CH)], dst_v[b],
                         idsem[b])

    def wait_dst(j, b):
        pltpu.make_async_copy(dst_hbm.at[pl.ds(e0 + j * _CH, _CH)], dst_v[b],
                              idsem[b]).wait()

    def gather(b):
        pltpu.async_copy(h_hbm.at[src_v[b]], msg_v[b], gsem[b])

    def wait_gather(b):
        pltpu.make_async_copy(h_hbm.at[src_v[b]], msg_v[b], gsem[b]).wait()

    def scatter(b):
        pltpu.async_copy(msg_v[b], agg_sh.at[dst_v[b]], ssem[b], add=True)

    def wait_scatter(b):
        pltpu.make_async_copy(msg_v[b], agg_sh.at[dst_v[b]], ssem[b]).wait()

    for b in range(_B):
        load_src(b, b)
        load_dst(b, b)
    pltpu.sync_copy(zblk_hbm, m0_v)
    _zero_my_slice(m0_v, agg_sh, s)
    plsc.subcore_barrier()

    def group(g, carry):
        for b in range(_B):
            j = _B * g + b

            @pl.when(g > 0)
            def _(b=b, j=j):
                wait_scatter(b)
                load_dst(j, b)

            wait_src(j, b)
            gather(b)
        for b in range(_B):
            j = _B * g + b
            wait_gather(b)
            wait_dst(j, b)
            scatter(b)
            if b == 0:
                load_src(j + _B, b)
            else:
                @pl.when(g < _NGRP - 1)
                def _(b=b, j=j):
                    load_src(j + _B, b)
        return carry

    lax.fori_loop(0, _NGRP, group, 0)

    last = _NCHUNK - 1
    wait_scatter(0)
    wait_src(last, 0)
    gather(0)
    load_dst(last, 0)
    wait_gather(0)
    wait_dst(last, 0)
    scatter(0)
    wait_scatter(0)
    for b in range(1, _B):
        wait_scatter(b)
    plsc.subcore_barrier()
    _flush_my_slice(agg_sh, out_hbm, c, s)


_mp_sc = pl.kernel(
    _mp_body,
    out_type=jax.ShapeDtypeStruct((_NC, _N, _H), jnp.float32),
    mesh=plsc.VectorSubcoreMesh(core_axis_name="c", subcore_axis_name="s"),
    scratch_types=(
        [pltpu.VMEM((_CH,), jnp.int32)] * (2 * _B)
        + [pltpu.VMEM((_CH, _H), jnp.float32)] * _B
        + [pltpu.VMEM_SHARED((_N, _H), jnp.float32)]
        + [pltpu.SemaphoreType.DMA] * (4 * _B)
    ),
)


def _deg_body(dst_hbm, zblk_hbm, oblk_hbm, out_hbm,
              dst0_v, dst1_v, buf_v, agg_sh,
              idsem0, idsem1, ssem0, ssem1):
    c = lax.axis_index("c")
    s = lax.axis_index("s")
    wid = c * _NS + s
    e0 = wid * _EPW

    def load_dst(j, buf, sem):
        pltpu.async_copy(dst_hbm.at[pl.ds(e0 + j * _CH, _CH)], buf, sem)

    def wait_dst(j, buf, sem):
        pltpu.make_async_copy(dst_hbm.at[pl.ds(e0 + j * _CH, _CH)], buf,
                              sem).wait()

    def scatter(buf_idx, sem):
        pltpu.async_copy(buf_v, agg_sh.at[buf_idx], sem, add=True)

    def wait_scatter(buf_idx, sem):
        pltpu.make_async_copy(buf_v, agg_sh.at[buf_idx], sem).wait()

    load_dst(0, dst0_v, idsem0)
    load_dst(1, dst1_v, idsem1)
    pltpu.sync_copy(zblk_hbm, buf_v)
    _zero_my_slice(buf_v, agg_sh, s)
    plsc.subcore_barrier()
    pltpu.sync_copy(oblk_hbm, buf_v)

    def pair(i, carry):
        j0 = 2 * i
        j1 = j0 + 1
        wait_dst(j0, dst0_v, idsem0)
        scatter(dst0_v, ssem0)
        wait_scatter(dst0_v, ssem0)
        load_dst(j0 + 2, dst0_v, idsem0)
        wait_dst(j1, dst1_v, idsem1)
        scatter(dst1_v, ssem1)
        wait_scatter(dst1_v, ssem1)

        @pl.when(i < _NPAIR - 1)
        def _():
            load_dst(j1 + 2, dst1_v, idsem1)

        return carry

    lax.fori_loop(0, _NPAIR, pair, 0)

    last = _NCHUNK - 1
    wait_dst(last, dst0_v, idsem0)
    scatter(dst0_v, ssem0)
    wait_scatter(dst0_v, ssem0)
    plsc.subcore_barrier()
    _flush_my_slice(agg_sh, out_hbm, c, s)


_deg_sc = pl.kernel(
    _deg_body,
    out_type=jax.ShapeDtypeStruct((_NC, _N, _H), jnp.float32),
    mesh=plsc.VectorSubcoreMesh(core_axis_name="c", subcore_axis_name="s"),
    scratch_types=[
        pltpu.VMEM((_CH,), jnp.int32),
        pltpu.VMEM((_CH,), jnp.int32),
        pltpu.VMEM((_CH, _H), jnp.float32),
        pltpu.VMEM_SHARED((_N, _H), jnp.float32),
        pltpu.SemaphoreType.DMA,
        pltpu.SemaphoreType.DMA,
        pltpu.SemaphoreType.DMA,
        pltpu.SemaphoreType.DMA,
    ],
)


def _enc_body(x_ref, w_ref, b_ref, o_ref):
    o_ref[...] = jnp.dot(x_ref[...], w_ref[...],
                         preferred_element_type=jnp.float32) + b_ref[...]


def _enc(x, w, b):
    return pl.pallas_call(
        _enc_body,
        out_shape=jax.ShapeDtypeStruct((_N, _H), jnp.float32),
    )(x, w, b.reshape(1, _H))


def _gin_body(h_ref, p_ref, w_ref, b_ref, g_ref, be_ref, o_ref):
    z = h_ref[...] + p_ref[0] + p_ref[1]
    r = jax.nn.relu(jnp.dot(z, w_ref[...],
                            preferred_element_type=jnp.float32) + b_ref[...])
    mu = jnp.mean(r, axis=0, keepdims=True)
    var = jnp.mean((r - mu) ** 2, axis=0, keepdims=True)
    o_ref[...] = (r - mu) / jnp.sqrt(var + 1e-5) * g_ref[...] + be_ref[...]


def _gin(h, parts, w, b, g, be):
    return pl.pallas_call(
        _gin_body,
        out_shape=jax.ShapeDtypeStruct((_N, _H), jnp.float32),
    )(h, parts, w, b.reshape(1, _H), g.reshape(1, _H), be.reshape(1, _H))


def _appnp_setup_body(dp_ref, h_ref, isd_ref, sn_ref, hs_ref):
    deg = dp_ref[0][:, :1] + dp_ref[1][:, :1] + 1.0
    isd = 1.0 / jnp.sqrt(deg)
    isd_ref[...] = isd
    sn_ref[...] = 1.0 / deg
    hs_ref[...] = h_ref[...] * isd


def _appnp_setup(deg_parts, h):
    return pl.pallas_call(
        _appnp_setup_body,
        out_shape=(
            jax.ShapeDtypeStruct((_N, 1), jnp.float32),
            jax.ShapeDtypeStruct((_N, 1), jnp.float32),
            jax.ShapeDtypeStruct((_N, _H), jnp.float32),
        ),
    )(deg_parts, h)


def _appnp_update_body(p_ref, h_ref, h0_ref, isd_ref, sn_ref, hn_ref, hsn_ref):
    agg = p_ref[0] + p_ref[1]
    prop = agg * isd_ref[...] + h_ref[...] * sn_ref[...]
    hn = (1.0 - _ALPHA) * prop + _ALPHA * h0_ref[...]
    hn_ref[...] = hn
    hsn_ref[...] = hn * isd_ref[...]


def _appnp_update(parts, h, h0, isd, sn):
    return pl.pallas_call(
        _appnp_update_body,
        out_shape=(
            jax.ShapeDtypeStruct((_N, _H), jnp.float32),
            jax.ShapeDtypeStruct((_N, _H), jnp.float32),
        ),
    )(parts, h, h0, isd, sn)


def _pool_body(h_ref, b_ref, w_ref, bo_ref, o_ref):
    gids = lax.broadcasted_iota(jnp.int32, (1, _G), 1)
    onehot = (b_ref[...] == gids).astype(jnp.float32)
    sums = lax.dot_general(onehot, h_ref[...], (((0,), (0,)), ((), ())),
                           preferred_element_type=jnp.float32)
    counts = jnp.sum(onehot, axis=0, keepdims=True)
    pooled = sums / jnp.maximum(counts, 1.0).T
    o_ref[...] = jnp.dot(pooled, w_ref[...],
                         preferred_element_type=jnp.float32) + bo_ref[...]


def _pool_out(h, batch, w, b):
    return pl.pallas_call(
        _pool_body,
        out_shape=jax.ShapeDtypeStruct((_G, _H), jnp.float32),
    )(h, batch.reshape(_N, 1), w, b.reshape(1, _H))


def kernel(x, edge_index, batch, W_enc, b_enc, W_layers, b_layers,
           gamma, beta, W_out, b_out):
    src = edge_index[0]
    dst = edge_index[1]
    zblk = jnp.zeros((_CH, _H), jnp.float32)
    oblk = jnp.ones((_CH, _H), jnp.float32)

    deg_parts = _deg_sc(dst, zblk, oblk)
    h = _enc(x, W_enc, b_enc)
    for i in range(_L):
        parts = _mp_sc(h, src, dst, zblk)
        h = _gin(h, parts, W_layers[i], b_layers[i], gamma[i], beta[i])

    isd, sn, hs = _appnp_setup(deg_parts, h)
    h0 = h
    for _ in range(_K):
        parts = _mp_sc(hs, src, dst, zblk)
        h, hs = _appnp_update(parts, h, h0, isd, sn)

    return _pool_out(h, batch, W_out, b_out)
